# baseline (device time: 10871 ns/iter reference)
import jax
import jax.numpy as jnp
from jax import lax
from jax.experimental import pallas as pl
from jax.experimental.pallas import tpu as pltpu

N_DEV = 4


def _cmpex(v, j, up):
    n = v.shape[0]
    iota = lax.broadcasted_iota(jnp.int32, v.shape, 0)
    is_lo = (iota & j) == 0
    vv = v.reshape(n // (2 * j), 2 * j, v.shape[1])
    partner = jnp.roll(vv, j, axis=1).reshape(v.shape)
    mn = jnp.minimum(v, partner)
    mx = jnp.maximum(v, partner)
    return jnp.where(up == is_lo, mn, mx)


def _local_sort(v, asc):
    n = v.shape[0]
    iota = lax.broadcasted_iota(jnp.int32, v.shape, 0)
    k = 2
    while k <= n:
        up = ((iota & k) == 0) == asc
        j = k // 2
        while j >= 1:
            v = _cmpex(v, j, up)
            j //= 2
        k *= 2
    return v


def _local_merge(v, asc):
    j = v.shape[0] // 2
    while j >= 1:
        v = _cmpex(v, j, asc)
        j //= 2
    return v


def kernel(x):
    m, n = x.shape

    def body(x_ref, out_ref, send_ref, recv_ref, send_sems, recv_sems):
        d = lax.axis_index("i")

        barrier_sem = pltpu.get_barrier_semaphore()
        for e in range(1, N_DEV):
            pl.semaphore_signal(
                barrier_sem,
                inc=1,
                device_id=(d ^ e,),
                device_id_type=pl.DeviceIdType.MESH,
            )

        d_even = (d & 1) == 0
        d_lo_half = d < 2

        va = _local_sort(x_ref[:, :].astype(jnp.bfloat16), d_even)
        send_ref[:, :] = va

        pl.semaphore_wait(barrier_sem, N_DEV - 1)

        rdmas = {}
        for e in [2, 1, 3]:
            rdma = pltpu.make_async_remote_copy(
                src_ref=send_ref,
                dst_ref=recv_ref.at[e - 1],
                send_sem=send_sems.at[e - 1],
                recv_sem=recv_sems.at[e - 1],
                device_id=(d ^ e,),
                device_id_type=pl.DeviceIdType.MESH,
            )
            rdma.start()
            rdmas[e] = rdma

        def _k512(u_s, u_sx1, s):
            be = d_even if (s & 1) == 0 else jnp.logical_not(d_even)
            blh = d_lo_half if (s & 2) == 0 else jnp.logical_not(d_lo_half)
            w = jnp.where(
                be == blh,
                jnp.minimum(u_s, u_sx1),
                jnp.maximum(u_s, u_sx1),
            )
            return _local_merge(w, blh)

        rdmas[1].wait_recv()
        u1 = recv_ref[0, :, :]
        vp = [None] * N_DEV
        vp[0] = _k512(va, u1, 0)
        vp[1] = _k512(u1, va, 1)
        rdmas[3].wait_recv()
        rdmas[2].wait_recv()
        u2 = recv_ref[1, :, :]
        u3 = recv_ref[2, :, :]
        vp[2] = _k512(u2, u3, 2)
        vp[3] = _k512(u3, u2, 3)

        a = jnp.where(
            d_lo_half, jnp.minimum(vp[0], vp[2]), jnp.maximum(vp[0], vp[2])
        )
        b = jnp.where(
            d_lo_half, jnp.minimum(vp[1], vp[3]), jnp.maximum(vp[1], vp[3])
        )
        g = jnp.where(d_even, jnp.minimum(a, b), jnp.maximum(a, b))
        g = _local_merge(g, True)
        out_ref[:, :] = g.astype(x_ref.dtype)

        for rdma in rdmas.values():
            rdma.wait_send()

    return pl.pallas_call(
        body,
        out_shape=jax.ShapeDtypeStruct((m, n), x.dtype),
        in_specs=[pl.BlockSpec(memory_space=pltpu.VMEM)],
        out_specs=pl.BlockSpec(memory_space=pltpu.VMEM),
        scratch_shapes=[
            pltpu.VMEM((m, n), jnp.bfloat16),
            pltpu.VMEM((N_DEV - 1, m, n), jnp.bfloat16),
            pltpu.SemaphoreType.DMA((N_DEV - 1,)),
            pltpu.SemaphoreType.DMA((N_DEV - 1,)),
        ],
        compiler_params=pltpu.CompilerParams(collective_id=0),
    )(x)


# device time: 9037 ns/iter; 1.2029x vs baseline; 1.2029x over previous
import jax
import jax.numpy as jnp
from jax import lax
from jax.experimental import pallas as pl
from jax.experimental.pallas import tpu as pltpu

N_DEV = 4


def _cmpex(v, j, up):
    n = v.shape[0]
    iota = lax.broadcasted_iota(jnp.int32, v.shape, 0)
    is_lo = (iota & j) == 0
    up_vals = jnp.concatenate([v[j:], v[:j]], axis=0)
    dn_vals = jnp.concatenate([v[n - j :], v[: n - j]], axis=0)
    partner = jnp.where(is_lo, up_vals, dn_vals)
    mn = jnp.minimum(v, partner)
    mx = jnp.maximum(v, partner)
    return jnp.where(up == is_lo, mn, mx)


def _local_sort(v, asc):
    n = v.shape[0]
    iota = lax.broadcasted_iota(jnp.int32, v.shape, 0)
    k = 2
    while k <= n:
        up = ((iota & k) == 0) == asc
        j = k // 2
        while j >= 1:
            v = _cmpex(v, j, up)
            j //= 2
        k *= 2
    return v


def _local_merge(v, asc):
    j = v.shape[0] // 2
    while j >= 1:
        v = _cmpex(v, j, asc)
        j //= 2
    return v


def kernel(x):
    m, n = x.shape

    def body(x_ref, out_ref, send_ref, recv_ref, send_sems, recv_sems):
        d = lax.axis_index("i")

        barrier_sem = pltpu.get_barrier_semaphore()
        for e in range(1, N_DEV):
            pl.semaphore_signal(
                barrier_sem,
                inc=1,
                device_id=(d ^ e,),
                device_id_type=pl.DeviceIdType.MESH,
            )

        d_even = (d & 1) == 0
        d_lo_half = d < 2

        va = _local_sort(x_ref[:, :].astype(jnp.bfloat16), d_even)
        send_ref[:, :] = va

        pl.semaphore_wait(barrier_sem, N_DEV - 1)

        rdmas = {}
        for e in [2, 1, 3]:
            rdma = pltpu.make_async_remote_copy(
                src_ref=send_ref,
                dst_ref=recv_ref.at[e - 1],
                send_sem=send_sems.at[e - 1],
                recv_sem=recv_sems.at[e - 1],
                device_id=(d ^ e,),
                device_id_type=pl.DeviceIdType.MESH,
            )
            rdma.start()
            rdmas[e] = rdma

        def _k512(u_s, u_sx1, s):
            be = d_even if (s & 1) == 0 else jnp.logical_not(d_even)
            blh = d_lo_half if (s & 2) == 0 else jnp.logical_not(d_lo_half)
            w = jnp.where(
                be == blh,
                jnp.minimum(u_s, u_sx1),
                jnp.maximum(u_s, u_sx1),
            )
            return _local_merge(w, blh)

        rdmas[1].wait_recv()
        u1 = recv_ref[0, :, :]
        vp = [None] * N_DEV
        vp[0] = _k512(va, u1, 0)
        vp[1] = _k512(u1, va, 1)
        rdmas[3].wait_recv()
        rdmas[2].wait_recv()
        u2 = recv_ref[1, :, :]
        u3 = recv_ref[2, :, :]
        vp[2] = _k512(u2, u3, 2)
        vp[3] = _k512(u3, u2, 3)

        a = jnp.where(
            d_lo_half, jnp.minimum(vp[0], vp[2]), jnp.maximum(vp[0], vp[2])
        )
        b = jnp.where(
            d_lo_half, jnp.minimum(vp[1], vp[3]), jnp.maximum(vp[1], vp[3])
        )
        g = jnp.where(d_even, jnp.minimum(a, b), jnp.maximum(a, b))
        g = _local_merge(g, True)
        out_ref[:, :] = g.astype(x_ref.dtype)

        for rdma in rdmas.values():
            rdma.wait_send()

    return pl.pallas_call(
        body,
        out_shape=jax.ShapeDtypeStruct((m, n), x.dtype),
        in_specs=[pl.BlockSpec(memory_space=pltpu.VMEM)],
        out_specs=pl.BlockSpec(memory_space=pltpu.VMEM),
        scratch_shapes=[
            pltpu.VMEM((m, n), jnp.bfloat16),
            pltpu.VMEM((N_DEV - 1, m, n), jnp.bfloat16),
            pltpu.SemaphoreType.DMA((N_DEV - 1,)),
            pltpu.SemaphoreType.DMA((N_DEV - 1,)),
        ],
        compiler_params=pltpu.CompilerParams(collective_id=0),
    )(x)
